# baseline (speedup 1.0000x reference)
import functools

import jax
import jax.numpy as jnp
from jax import lax
from jax.experimental import pallas as pl
from jax.experimental.pallas import tpu as pltpu
from jax.experimental.pallas import tpu_sc as plsc

_NC = 2
_NS = 16
_NW = _NC * _NS
_CH = 128
_K = 8


def _transpose_block(tin_s, tout_s, io, d_dim, nq):
    for d in range(d_dim):
        for q in range(nq):
            vec = tin_s[d, pl.ds(q * 16, 16)]
            plsc.store_scatter(tout_s, [io + (q * 16 * d_dim + d)], vec)


@functools.partial(jax.jit, static_argnums=(2, 3))
def _sc_table_to_rowmajor(tbl_t, tail_pad, v, d_dim):
    n_full = v // _CH
    mesh = plsc.VectorSubcoreMesh(core_axis_name="c", subcore_axis_name="s")
    n_t = n_full // _NW + (1 if n_full % _NW else 0)

    @functools.partial(
        pl.kernel,
        out_type=jax.ShapeDtypeStruct((v * d_dim,), jnp.float32),
        mesh=mesh,
        compiler_params=pltpu.CompilerParams(
            use_tc_tiling_on_sc=True, needs_layout_passes=False
        ),
        scratch_types=[
            pltpu.VMEM((d_dim, _CH), jnp.float32),
            pltpu.VMEM((d_dim, _CH), jnp.float32),
            pltpu.VMEM((d_dim, _CH), jnp.float32),
            pltpu.VMEM((d_dim, _CH), jnp.float32),
            pltpu.VMEM((_CH * d_dim,), jnp.float32),
            pltpu.VMEM((_CH * d_dim,), jnp.float32),
            pltpu.VMEM((_CH * d_dim,), jnp.float32),
            pltpu.VMEM((_CH * d_dim,), jnp.float32),
            pltpu.SemaphoreType.DMA,
            pltpu.SemaphoreType.DMA,
            pltpu.SemaphoreType.DMA,
            pltpu.SemaphoreType.DMA,
            pltpu.SemaphoreType.DMA,
            pltpu.SemaphoreType.DMA,
            pltpu.SemaphoreType.DMA,
            pltpu.SemaphoreType.DMA,
        ],
    )
    def ka(tbl_hbm, tail_hbm, out_hbm, tin0, tin1, tin2, tin3,
           tout0, tout1, tout2, tout3, g0, g1, g2, g3, s0, s1, s2, s3):
        wid = lax.axis_index("s") * _NC + lax.axis_index("c")
        tin = (tin0, tin1, tin2, tin3)
        tout = (tout0, tout1, tout2, tout3)
        gsems = (g0, g1, g2, g3)
        ssems = (s0, s1, s2, s3)
        io = jnp.arange(16, dtype=jnp.int32) * d_dim
        blk = _CH * d_dim

        def bid(t):
            return t * _NW + wid

        def valid(t):
            bt = bid(t)
            return jnp.logical_and(t >= 0, bt < n_full)

        def fire(t, slot):
            @pl.when(valid(t))
            def _():
                pltpu.async_copy(
                    tbl_hbm.at[:, pl.ds(bid(t) * _CH, _CH)],
                    tin[slot], gsems[slot],
                )

        def wait_in(t, slot):
            @pl.when(valid(t))
            def _():
                pltpu.make_async_copy(
                    tbl_hbm.at[:, pl.ds(bid(t) * _CH, _CH)],
                    tin[slot], gsems[slot],
                ).wait()

        def compute(t, slot):
            @pl.when(valid(t))
            def _():
                _transpose_block(tin[slot], tout[slot], io, d_dim, 8)

        def store(t, slot):
            @pl.when(valid(t))
            def _():
                pltpu.async_copy(
                    tout[slot], out_hbm.at[pl.ds(bid(t) * blk, blk)],
                    ssems[slot],
                )

        def wait_store(t, slot):
            @pl.when(valid(t))
            def _():
                pltpu.make_async_copy(
                    tout[slot], out_hbm.at[pl.ds(bid(t) * blk, blk)],
                    ssems[slot],
                ).wait()

        def step(t, slot):
            wait_store(t - 4, slot)
            fire(t + 3, (slot + 3) % 4)
            wait_in(t, slot)
            compute(t, slot)
            store(t, slot)

        fire(0, 0)
        fire(1, 1)
        fire(2, 2)
        n_quad = n_t // 4

        @pl.loop(0, n_quad)
        def _(p):
            t0 = 4 * p
            step(t0, 0)
            step(t0 + 1, 1)
            step(t0 + 2, 2)
            step(t0 + 3, 3)

        for r in range(n_quad * 4, n_t):
            step(r, r % 4)
        for r in range(n_t - 4, n_t):
            wait_store(r, r % 4)

        rem = v - n_full * _CH
        if rem:
            @pl.when(wid == 0)
            def _():
                pltpu.sync_copy(tail_hbm, tin0)
                for d in range(d_dim):
                    for q in range(rem // 16):
                        vec = tin0[d, pl.ds(q * 16, 16)]
                        plsc.store_scatter(
                            tout0, [io + (q * 16 * d_dim + d)], vec
                        )
                pltpu.sync_copy(
                    tout0.at[pl.ds(0, rem * d_dim)],
                    out_hbm.at[pl.ds(n_full * blk, rem * d_dim)],
                )

    return ka(tbl_t, tail_pad)


@functools.partial(jax.jit, static_argnums=(2, 3, 4))
def _sc_embedding_lookup(idx, table, n_rows, per_w, n_ch):
    d = table.shape[1]
    n_mc = n_ch // _K
    mc = _K * _CH
    mesh = plsc.VectorSubcoreMesh(core_axis_name="c", subcore_axis_name="s")
    n_pair = n_mc // 2

    @functools.partial(
        pl.kernel,
        out_type=jax.ShapeDtypeStruct((n_rows, d), table.dtype),
        mesh=mesh,
        compiler_params=pltpu.CompilerParams(use_tc_tiling_on_sc=False),
        scratch_types=[
            pltpu.VMEM((n_ch, _CH), jnp.int32),
            pltpu.VMEM((2, mc, d), table.dtype),
            pltpu.SemaphoreType.DMA,
            pltpu.SemaphoreType.DMA,
            pltpu.SemaphoreType.DMA,
            pltpu.SemaphoreType.DMA,
        ],
    )
    def emb(idx_hbm, table_hbm, out_hbm, idx_v, rows_v, g0, g1, s0, s1):
        wid = lax.axis_index("s") * _NC + lax.axis_index("c")
        base = wid * per_w
        gsems = (g0, g1)
        ssems = (s0, s1)
        pltpu.sync_copy(idx_hbm.at[wid], idx_v)

        def fire(m, slot):
            for b in range(_K):
                pltpu.async_copy(
                    table_hbm.at[idx_v.at[m * _K + b]],
                    rows_v.at[slot].at[pl.ds(b * _CH, _CH)],
                    gsems[slot],
                )

        def drain(m, slot):
            for b in range(_K):
                pltpu.make_async_copy(
                    table_hbm.at[idx_v.at[m * _K + b]],
                    rows_v.at[slot].at[pl.ds(b * _CH, _CH)],
                    gsems[slot],
                ).wait()

        def store(m, slot):
            pltpu.async_copy(
                rows_v.at[slot], out_hbm.at[pl.ds(base + m * mc, mc)],
                ssems[slot],
            )

        def wait_store(m, slot):
            pltpu.make_async_copy(
                rows_v.at[slot], out_hbm.at[pl.ds(base + m * mc, mc)],
                ssems[slot],
            ).wait()

        fire(0, 0)

        @pl.loop(0, n_pair)
        def _(p):
            m0 = 2 * p
            @pl.when(p >= 1)
            def _():
                wait_store(m0 - 1, 1)

            fire(m0 + 1, 1)
            drain(m0, 0)
            store(m0, 0)
            wait_store(m0, 0)

            @pl.when(m0 + 2 < n_mc)
            def _():
                fire(m0 + 2, 0)

            drain(m0 + 1, 1)
            store(m0 + 1, 1)

        if n_mc % 2:
            mt = n_mc - 1
            wait_store(mt - 1, 1)
            drain(mt, 0)
            store(mt, 0)
            wait_store(mt, 0)
        else:
            wait_store(n_mc - 1, 1)

    return emb(idx, table)


def kernel(inputs, table):
    b, f = inputs.shape
    v, d = table.shape
    n = b * f
    per_w = n // _NW
    n_ch = per_w // _CH
    idx = inputs.reshape(_NW, n_ch, _CH).astype(jnp.int32)
    tbl_t = table.T
    n_full = v // _CH
    rem = v - n_full * _CH
    tail_pad = jnp.pad(tbl_t[:, n_full * _CH:], ((0, 0), (0, _CH - rem)))
    tlin = _sc_table_to_rowmajor(tbl_t, tail_pad, v, d).reshape(v, d)
    out = _sc_embedding_lookup(idx, tlin, n, per_w, n_ch)
    return out.reshape(b, f, d)


# speedup vs baseline: 1.1808x; 1.1808x over previous
import functools

import jax
import jax.numpy as jnp
from jax import lax
from jax.experimental import pallas as pl
from jax.experimental.pallas import tpu as pltpu
from jax.experimental.pallas import tpu_sc as plsc

_NC = 2
_NS = 16
_NW = _NC * _NS
_CH = 128
_K = 8


@functools.partial(jax.jit, static_argnums=(2, 3, 4))
def _sc_embedding_lookup(idx, table, n_rows, per_w, n_ch):
    d = table.shape[1]
    n_mc = n_ch // _K
    mc = _K * _CH
    mesh = plsc.VectorSubcoreMesh(core_axis_name="c", subcore_axis_name="s")
    n_pair = n_mc // 2

    @functools.partial(
        pl.kernel,
        out_type=jax.ShapeDtypeStruct((n_rows, d), table.dtype),
        mesh=mesh,
        compiler_params=pltpu.CompilerParams(use_tc_tiling_on_sc=False),
        scratch_types=[
            pltpu.VMEM((n_ch, _CH), jnp.int32),
            pltpu.VMEM((2, mc, d), table.dtype),
            pltpu.SemaphoreType.DMA,
            pltpu.SemaphoreType.DMA,
            pltpu.SemaphoreType.DMA,
            pltpu.SemaphoreType.DMA,
        ],
    )
    def emb(idx_hbm, table_hbm, out_hbm, idx_v, rows_v, g0, g1, s0, s1):
        wid = lax.axis_index("s") * _NC + lax.axis_index("c")
        base = wid * per_w
        gsems = (g0, g1)
        ssems = (s0, s1)
        pltpu.sync_copy(idx_hbm.at[wid], idx_v)

        def fire(m, slot):
            for b in range(_K):
                pltpu.async_copy(
                    table_hbm.at[idx_v.at[m * _K + b]],
                    rows_v.at[slot].at[pl.ds(b * _CH, _CH)],
                    gsems[slot],
                )

        def drain(m, slot):
            for b in range(_K):
                pltpu.make_async_copy(
                    table_hbm.at[idx_v.at[m * _K + b]],
                    rows_v.at[slot].at[pl.ds(b * _CH, _CH)],
                    gsems[slot],
                ).wait()

        def store(m, slot):
            pltpu.async_copy(
                rows_v.at[slot], out_hbm.at[pl.ds(base + m * mc, mc)],
                ssems[slot],
            )

        def wait_store(m, slot):
            pltpu.make_async_copy(
                rows_v.at[slot], out_hbm.at[pl.ds(base + m * mc, mc)],
                ssems[slot],
            ).wait()

        fire(0, 0)

        @pl.loop(0, n_pair)
        def _(p):
            m0 = 2 * p
            @pl.when(p >= 1)
            def _():
                wait_store(m0 - 1, 1)

            fire(m0 + 1, 1)
            drain(m0, 0)
            store(m0, 0)
            wait_store(m0, 0)

            @pl.when(m0 + 2 < n_mc)
            def _():
                fire(m0 + 2, 0)

            drain(m0 + 1, 1)
            store(m0 + 1, 1)

        if n_mc % 2:
            mt = n_mc - 1
            wait_store(mt - 1, 1)
            drain(mt, 0)
            store(mt, 0)
            wait_store(mt, 0)
        else:
            wait_store(n_mc - 1, 1)

    return emb(idx, table)


def kernel(inputs, table):
    b, f = inputs.shape
    v, d = table.shape
    n = b * f
    per_w = n // _NW
    n_ch = per_w // _CH
    idx = inputs.reshape(_NW, n_ch, _CH).astype(jnp.int32)
    out = _sc_embedding_lookup(idx, table, n, per_w, n_ch)
    return out.reshape(b, f, d)
